# baseline (speedup 1.0000x reference)
import functools

import jax
import jax.numpy as jnp
from jax import lax
from jax.experimental import pallas as pl
from jax.experimental.pallas import tpu as pltpu
from jax.experimental.pallas import tpu_sc as plsc

N = 4096 * 26
D = 64
_PW = 128

_NUM_CORES = 2
_NUM_SUBCORES = 16
_NW = _NUM_CORES * _NUM_SUBCORES
_B_PER_W = N // _NW
_NBUF = 4
_CHUNK = 208
_NCHUNK = _B_PER_W // _CHUNK


@functools.partial(
    pl.kernel,
    mesh=plsc.VectorSubcoreMesh(core_axis_name="c", subcore_axis_name="s"),
    out_type=jax.ShapeDtypeStruct((N, _PW), jnp.float32),
    scratch_types=[
        pltpu.VMEM((_B_PER_W,), jnp.int32),
        pltpu.VMEM((_CHUNK, _PW), jnp.float32),
        pltpu.VMEM((_CHUNK, _PW), jnp.float32),
        pltpu.VMEM((_CHUNK, _PW), jnp.float32),
        pltpu.VMEM((_CHUNK, _PW), jnp.float32),
        pltpu.SemaphoreType.DMA,
        pltpu.SemaphoreType.DMA,
        pltpu.SemaphoreType.DMA,
        pltpu.SemaphoreType.DMA,
        pltpu.SemaphoreType.DMA,
        pltpu.SemaphoreType.DMA,
        pltpu.SemaphoreType.DMA,
        pltpu.SemaphoreType.DMA,
    ],
    compiler_params=pltpu.CompilerParams(use_tc_tiling_on_sc=True),
)
def _gather_kernel(
    table_hbm,
    idx_hbm,
    out_hbm,
    idx_v,
    buf0,
    buf1,
    buf2,
    buf3,
    gsem0,
    gsem1,
    gsem2,
    gsem3,
    wsem0,
    wsem1,
    wsem2,
    wsem3,
):
    wid = lax.axis_index("s") * _NUM_CORES + lax.axis_index("c")
    base = wid * _B_PER_W
    pltpu.sync_copy(idx_hbm.at[pl.ds(base, _B_PER_W)], idx_v)

    bufs = (buf0, buf1, buf2, buf3)
    gsems = (gsem0, gsem1, gsem2, gsem3)
    wsems = (wsem0, wsem1, wsem2, wsem3)
    gathers = []
    writebacks = []
    for c in range(_NCHUNK):
        b = c % _NBUF
        if c >= 2:
            gathers[c - 2].wait()
            writebacks.append(
                pltpu.async_copy(
                    bufs[(c - 2) % _NBUF],
                    out_hbm.at[pl.ds(base + (c - 2) * _CHUNK, _CHUNK)],
                    wsems[(c - 2) % _NBUF],
                )
            )
        if c >= _NBUF:
            writebacks[c - _NBUF].wait()
        gathers.append(
            pltpu.async_copy(
                table_hbm.at[idx_v.at[pl.ds(c * _CHUNK, _CHUNK)]], bufs[b], gsems[b]
            )
        )
    for c in range(_NCHUNK - 2, _NCHUNK):
        b = c % _NBUF
        gathers[c].wait()
        writebacks.append(
            pltpu.async_copy(
                bufs[b], out_hbm.at[pl.ds(base + c * _CHUNK, _CHUNK)], wsems[b]
            )
        )
    for c in range(_NCHUNK - _NBUF, _NCHUNK):
        writebacks[c].wait()


def kernel(local_embs, lengths, unbucketize_permute_tensor):
    del lengths
    idx = unbucketize_permute_tensor.astype(jnp.int32)
    padded = jnp.pad(local_embs, ((0, 0), (0, _PW - D)))
    return _gather_kernel(padded, idx)[:, :D]


# speedup vs baseline: 1.0121x; 1.0121x over previous
import functools

import jax
import jax.numpy as jnp
from jax import lax
from jax.experimental import pallas as pl
from jax.experimental.pallas import tpu as pltpu
from jax.experimental.pallas import tpu_sc as plsc

N = 4096 * 26
D = 64
_PW = 128

_NUM_CORES = 2
_NUM_SUBCORES = 16
_NW = _NUM_CORES * _NUM_SUBCORES
_B_PER_W = N // _NW
_CHUNK = 416
_NCHUNK = _B_PER_W // _CHUNK


@functools.partial(
    pl.kernel,
    mesh=plsc.VectorSubcoreMesh(core_axis_name="c", subcore_axis_name="s"),
    out_type=jax.ShapeDtypeStruct((N, _PW), jnp.float32),
    scratch_types=[
        pltpu.VMEM((_B_PER_W,), jnp.int32),
        pltpu.VMEM((_CHUNK, _PW), jnp.float32),
        pltpu.VMEM((_CHUNK, _PW), jnp.float32),
        pltpu.SemaphoreType.DMA,
        pltpu.SemaphoreType.DMA,
    ],
    compiler_params=pltpu.CompilerParams(use_tc_tiling_on_sc=True),
)
def _gather_kernel(table_hbm, idx_hbm, out_hbm, idx_v, buf0, buf1, sem0, sem1):
    wid = lax.axis_index("s") * _NUM_CORES + lax.axis_index("c")
    base = wid * _B_PER_W
    pltpu.sync_copy(idx_hbm.at[pl.ds(base, _B_PER_W)], idx_v)

    bufs = (buf0, buf1)
    sems = (sem0, sem1)
    copies = []
    for c in range(_NCHUNK):
        b = c % 2
        if c >= 2:
            copies[c - 2].wait()
            pltpu.sync_copy(bufs[b], out_hbm.at[pl.ds(base + (c - 2) * _CHUNK, _CHUNK)])
        cp = pltpu.async_copy(
            table_hbm.at[idx_v.at[pl.ds(c * _CHUNK, _CHUNK)]], bufs[b], sems[b]
        )
        copies.append(cp)
    for c in range(_NCHUNK - 2, _NCHUNK):
        b = c % 2
        copies[c].wait()
        pltpu.sync_copy(bufs[b], out_hbm.at[pl.ds(base + c * _CHUNK, _CHUNK)])


def kernel(local_embs, lengths, unbucketize_permute_tensor):
    del lengths
    idx = unbucketize_permute_tensor.astype(jnp.int32)
    padded = jnp.pad(local_embs, ((0, 0), (0, _PW - D)))
    return _gather_kernel(padded, idx)[:, :D]
